# baseline (device time: 346259 ns/iter reference)
import functools

import jax
import jax.numpy as jnp
from jax import lax
from jax.experimental import pallas as pl
from jax.experimental.pallas import tpu as pltpu

N_DEV = 32
E_PER = 4
CAP = 48
BLK = E_PER * CAP
N_SLOTS = N_DEV * BLK


def kernel(x, router_W, route_idx, expert_W, shared_W):
    n_tok, d_model = x.shape
    e_per, _, d_ff = expert_W.shape
    assert e_per == E_PER

    scores = x @ router_W
    scores = scores - scores.max(axis=-1, keepdims=True)
    probs = jnp.exp(scores)
    probs = probs / probs.sum(axis=-1, keepdims=True)

    n_exp = router_W.shape[1]
    e_col = route_idx.astype(jnp.int32)
    onehot = (e_col == jnp.arange(n_exp, dtype=jnp.int32)[None, :])
    p_sel = jnp.sum(probs * onehot, axis=1, keepdims=True)

    oh_i = onehot.astype(jnp.int32)
    cum = jnp.cumsum(oh_i, axis=0)
    within = jnp.sum(oh_i * cum, axis=1).astype(jnp.int32) - 1
    e_global = e_col[:, 0]
    sid = jnp.where(
        within < CAP, e_global * CAP + within, N_SLOTS
    ).astype(jnp.int32)

    x_bf = x.astype(jnp.bfloat16)
    p_row = p_sel.reshape(1, n_tok)
    sid_row = sid.reshape(1, n_tok)
    sid_col = sid.reshape(n_tok, 1)
    w_bf = expert_W.astype(jnp.bfloat16)
    sw_bf = shared_W.astype(jnp.bfloat16)

    def body(x_ref, p_ref, sidr_ref, sidc_ref, w_ref, sw_ref, out_ref,
             s_ref, r_ref, rout_ref, rin_ref, sx, rx, sy, ry):
        my = lax.axis_index("i")

        barrier_sem = pltpu.get_barrier_semaphore()
        for d in range(1, N_DEV):
            pl.semaphore_signal(
                barrier_sem, inc=1,
                device_id=(lax.rem(my + d, N_DEV),),
                device_id_type=pl.DeviceIdType.MESH,
            )
        pl.semaphore_wait(barrier_sem, N_DEV - 1)

        sid_r = sidr_ref[...]
        pv = p_ref[...]
        xv = x_ref[...]

        def dispatch_block(dst, to_ref):
            rows = lax.broadcasted_iota(
                jnp.int32, (BLK, n_tok), 0) + dst * BLK
            g = jnp.where(rows == sid_r, pv, 0.0).astype(jnp.bfloat16)
            to_ref[pl.ds(dst * BLK, BLK), :] = jnp.dot(
                g, xv, preferred_element_type=jnp.float32
            ).astype(jnp.bfloat16)

        x_rdmas = []
        for d in range(1, N_DEV):
            peer = lax.rem(my + d, N_DEV)
            dispatch_block(peer, s_ref)
            rdma = pltpu.make_async_remote_copy(
                src_ref=s_ref.at[pl.ds(peer * BLK, BLK)],
                dst_ref=r_ref.at[pl.ds(my * BLK, BLK)],
                send_sem=sx.at[peer],
                recv_sem=rx.at[my],
                device_id=(peer,),
                device_id_type=pl.DeviceIdType.MESH,
            )
            rdma.start()
            x_rdmas.append(rdma)
        dispatch_block(my, r_ref)

        n_tiles = 4
        tile = n_tok // n_tiles
        for t in range(n_tiles):
            rows = pl.ds(t * tile, tile)
            out_ref[rows, :] = jnp.dot(
                x_ref[rows, :], sw_ref[...],
                preferred_element_type=jnp.float32,
            )

        def expert_block(src, dst_ref):
            for e in range(E_PER):
                rows = pl.ds(src * BLK + e * CAP, CAP)
                y = jnp.dot(
                    r_ref[rows, :], w_ref[e],
                    preferred_element_type=jnp.float32,
                )
                dst_ref[rows, :] = y.astype(jnp.bfloat16)

        def combine_block(owner):
            for t in range(n_tiles):
                rows = pl.ds(t * tile, tile)
                sc = sidc_ref[rows, :]
                cols = lax.broadcasted_iota(
                    jnp.int32, (tile, BLK), 1) + owner * BLK
                gt = (cols == sc).astype(jnp.float32).astype(jnp.bfloat16)
                out_ref[rows, :] += jnp.dot(
                    gt, rin_ref[pl.ds(owner * BLK, BLK), :],
                    preferred_element_type=jnp.float32,
                )

        expert_block(my, rin_ref)
        combine_block(my)

        y_rdmas = []
        for d in range(1, N_DEV):
            peer = lax.rem(my + d, N_DEV)
            recv = pltpu.make_async_remote_copy(
                src_ref=s_ref.at[pl.ds(0, BLK)],
                dst_ref=r_ref.at[pl.ds(peer * BLK, BLK)],
                send_sem=sx.at[peer],
                recv_sem=rx.at[peer],
                device_id=(peer,),
                device_id_type=pl.DeviceIdType.MESH,
            )
            recv.wait_recv()
            expert_block(peer, rout_ref)
            back = pltpu.make_async_remote_copy(
                src_ref=rout_ref.at[pl.ds(peer * BLK, BLK)],
                dst_ref=rin_ref.at[pl.ds(my * BLK, BLK)],
                send_sem=sy.at[peer],
                recv_sem=ry.at[my],
                device_id=(peer,),
                device_id_type=pl.DeviceIdType.MESH,
            )
            back.start()
            y_rdmas.append(back)

        for rdma in x_rdmas:
            rdma.wait_send()
        for d in range(1, N_DEV):
            peer = lax.rem(my + d, N_DEV)
            recv = pltpu.make_async_remote_copy(
                src_ref=rout_ref.at[pl.ds(0, BLK)],
                dst_ref=rin_ref.at[pl.ds(peer * BLK, BLK)],
                send_sem=sy.at[peer],
                recv_sem=ry.at[peer],
                device_id=(peer,),
                device_id_type=pl.DeviceIdType.MESH,
            )
            recv.wait_recv()
            combine_block(peer)
        for rdma in y_rdmas:
            rdma.wait_send()

        @functools.partial(
            pl.run_scoped, exit_sem=pltpu.SemaphoreType.REGULAR
        )
        def _(exit_sem):
            for d in range(1, N_DEV):
                pl.semaphore_signal(
                    exit_sem, inc=1,
                    device_id=(lax.rem(my + d, N_DEV),),
                    device_id_type=pl.DeviceIdType.MESH,
                )
            pl.semaphore_wait(exit_sem, N_DEV - 1)

    return pl.pallas_call(
        body,
        out_shape=jax.ShapeDtypeStruct((n_tok, d_ff), jnp.float32),
        in_specs=[
            pl.BlockSpec(memory_space=pltpu.VMEM),
            pl.BlockSpec(memory_space=pltpu.VMEM),
            pl.BlockSpec(memory_space=pltpu.VMEM),
            pl.BlockSpec(memory_space=pltpu.VMEM),
            pl.BlockSpec(memory_space=pltpu.VMEM),
            pl.BlockSpec(memory_space=pltpu.VMEM),
        ],
        out_specs=pl.BlockSpec(memory_space=pltpu.VMEM),
        scratch_shapes=[
            pltpu.VMEM((N_SLOTS, d_model), jnp.bfloat16),
            pltpu.VMEM((N_SLOTS, d_model), jnp.bfloat16),
            pltpu.VMEM((N_SLOTS, d_ff), jnp.bfloat16),
            pltpu.VMEM((N_SLOTS, d_ff), jnp.bfloat16),
            pltpu.SemaphoreType.DMA((N_DEV,)),
            pltpu.SemaphoreType.DMA((N_DEV,)),
            pltpu.SemaphoreType.DMA((N_DEV,)),
            pltpu.SemaphoreType.DMA((N_DEV,)),
        ],
        compiler_params=pltpu.CompilerParams(
            collective_id=0,
            vmem_limit_bytes=100 * 1024 * 1024,
        ),
    )(x_bf, p_row, sid_row, sid_col, w_bf, sw_bf)


# device time: 278813 ns/iter; 1.2419x vs baseline; 1.2419x over previous
import functools

import jax
import jax.numpy as jnp
from jax import lax
from jax.experimental import pallas as pl
from jax.experimental.pallas import tpu as pltpu

N_DEV = 32
E_PER = 4
BLKC = 144
N_SLOTS = N_DEV * BLKC


def kernel(x, router_W, route_idx, expert_W, shared_W):
    n_tok, d_model = x.shape
    e_per, _, d_ff = expert_W.shape
    assert e_per == E_PER

    scores = x @ router_W
    scores = scores - scores.max(axis=-1, keepdims=True)
    probs = jnp.exp(scores)
    probs = probs / probs.sum(axis=-1, keepdims=True)

    n_exp = router_W.shape[1]
    e_col = route_idx.astype(jnp.int32)
    ar = jnp.arange(n_exp, dtype=jnp.int32)[None, :]
    onehot = e_col == ar
    p_sel = jnp.sum(probs * onehot, axis=1, keepdims=True)

    oh_i = onehot.astype(jnp.int32)
    cum = jnp.cumsum(oh_i, axis=0)
    within = jnp.sum(oh_i * cum, axis=1).astype(jnp.int32) - 1
    tot = cum[-1, :]
    own_col = e_col // E_PER

    lower_mask = (ar // E_PER == own_col) & (ar < e_col)
    lower = jnp.sum(tot[None, :] * lower_mask, axis=1).astype(jnp.int32)
    rank = lower + within
    sid = jnp.where(
        rank < BLKC, own_col[:, 0] * BLKC + rank, N_SLOTS
    ).astype(jnp.int32)

    ends = jnp.minimum(
        jnp.cumsum(tot.reshape(N_DEV, E_PER), axis=1), BLKC
    ).astype(jnp.int32)
    cnt_in = (
        jnp.zeros((N_DEV, 8, 128), jnp.int32)
        .at[:, 0, :E_PER].set(ends)
        .reshape(N_DEV * 8, 128)
    )

    x_bf = x.astype(jnp.bfloat16)
    p_row = p_sel.reshape(1, n_tok)
    sid_row = sid.reshape(1, n_tok)
    sid_col = sid.reshape(n_tok, 1)
    w_bf = expert_W.astype(jnp.bfloat16)
    sw_bf = shared_W.astype(jnp.bfloat16)

    def body(x_ref, p_ref, sidr_ref, sidc_ref, w_ref, sw_ref, cin_ref,
             out_ref, s_ref, r_ref, rout_ref, rin_ref, crcv_ref,
             sx, rx, sy, ry, sc_sem, rc_sem):
        my = lax.axis_index("i")

        barrier_sem = pltpu.get_barrier_semaphore()
        for d in range(1, N_DEV):
            pl.semaphore_signal(
                barrier_sem, inc=1,
                device_id=(lax.rem(my + d, N_DEV),),
                device_id_type=pl.DeviceIdType.MESH,
            )
        pl.semaphore_wait(barrier_sem, N_DEV - 1)

        sid_r = sidr_ref[...]
        pv = p_ref[...]
        xv = x_ref[...]

        def dispatch_block(dst, to_ref):
            rows = lax.broadcasted_iota(
                jnp.int32, (BLKC, n_tok), 0) + dst * BLKC
            g = jnp.where(rows == sid_r, pv, 0.0).astype(jnp.bfloat16)
            to_ref[pl.ds(dst * BLKC, BLKC), :] = jnp.dot(
                g, xv, preferred_element_type=jnp.float32
            ).astype(jnp.bfloat16)

        x_rdmas = []
        for d in range(1, N_DEV):
            peer = lax.rem(my + d, N_DEV)
            dispatch_block(peer, s_ref)
            rdma = pltpu.make_async_remote_copy(
                src_ref=s_ref.at[pl.ds(peer * BLKC, BLKC)],
                dst_ref=r_ref.at[pl.ds(my * BLKC, BLKC)],
                send_sem=sx.at[peer],
                recv_sem=rx.at[my],
                device_id=(peer,),
                device_id_type=pl.DeviceIdType.MESH,
            )
            rdma.start()
            crdma = pltpu.make_async_remote_copy(
                src_ref=cin_ref.at[pl.ds(peer * 8, 8)],
                dst_ref=crcv_ref.at[pl.ds(my * 8, 8)],
                send_sem=sc_sem.at[peer],
                recv_sem=rc_sem.at[my],
                device_id=(peer,),
                device_id_type=pl.DeviceIdType.MESH,
            )
            crdma.start()
            x_rdmas.append((rdma, crdma))
        dispatch_block(my, r_ref)
        crcv_ref[pl.ds(my * 8, 8), :] = cin_ref[pl.ds(my * 8, 8), :]

        n_tiles = 4
        tile = n_tok // n_tiles
        for t in range(n_tiles):
            rows = pl.ds(t * tile, tile)
            out_ref[rows, :] = jnp.dot(
                x_ref[rows, :], sw_ref[...],
                preferred_element_type=jnp.float32,
            )

        iota_col = lax.broadcasted_iota(jnp.int32, (BLKC, 1), 0)

        def expert_block(src, dst_ref):
            rblk = r_ref[pl.ds(src * BLKC, BLKC), :]
            acc = None
            for e in range(E_PER):
                if e == 0:
                    m_lo = iota_col >= 0
                else:
                    st = crcv_ref[pl.ds(src * 8, 1), pl.ds(e - 1, 1)]
                    m_lo = iota_col >= st
                en = crcv_ref[pl.ds(src * 8, 1), pl.ds(e, 1)]
                mask = (m_lo & (iota_col < en))
                mb = mask.astype(jnp.float32).astype(jnp.bfloat16)
                y = jnp.dot(
                    rblk * mb, w_ref[e],
                    preferred_element_type=jnp.float32,
                )
                acc = y if acc is None else acc + y
            dst_ref[pl.ds(src * BLKC, BLKC), :] = acc.astype(jnp.bfloat16)

        def combine_block(owner):
            for t in range(n_tiles):
                rows = pl.ds(t * tile, tile)
                sc = sidc_ref[rows, :]
                cols = lax.broadcasted_iota(
                    jnp.int32, (tile, BLKC), 1) + owner * BLKC
                gt = (cols == sc).astype(jnp.float32).astype(jnp.bfloat16)
                out_ref[rows, :] += jnp.dot(
                    gt, rin_ref[pl.ds(owner * BLKC, BLKC), :],
                    preferred_element_type=jnp.float32,
                )

        expert_block(my, rin_ref)
        combine_block(my)

        y_rdmas = []
        for d in range(1, N_DEV):
            peer = lax.rem(my + d, N_DEV)
            recv = pltpu.make_async_remote_copy(
                src_ref=s_ref.at[pl.ds(0, BLKC)],
                dst_ref=r_ref.at[pl.ds(peer * BLKC, BLKC)],
                send_sem=sx.at[peer],
                recv_sem=rx.at[peer],
                device_id=(peer,),
                device_id_type=pl.DeviceIdType.MESH,
            )
            recv.wait_recv()
            crecv = pltpu.make_async_remote_copy(
                src_ref=cin_ref.at[pl.ds(0, 8)],
                dst_ref=crcv_ref.at[pl.ds(peer * 8, 8)],
                send_sem=sc_sem.at[peer],
                recv_sem=rc_sem.at[peer],
                device_id=(peer,),
                device_id_type=pl.DeviceIdType.MESH,
            )
            crecv.wait_recv()
            expert_block(peer, rout_ref)
            back = pltpu.make_async_remote_copy(
                src_ref=rout_ref.at[pl.ds(peer * BLKC, BLKC)],
                dst_ref=rin_ref.at[pl.ds(my * BLKC, BLKC)],
                send_sem=sy.at[peer],
                recv_sem=ry.at[my],
                device_id=(peer,),
                device_id_type=pl.DeviceIdType.MESH,
            )
            back.start()
            y_rdmas.append(back)

        for rdma, crdma in x_rdmas:
            rdma.wait_send()
            crdma.wait_send()
        for d in range(1, N_DEV):
            peer = lax.rem(my + d, N_DEV)
            recv = pltpu.make_async_remote_copy(
                src_ref=rout_ref.at[pl.ds(0, BLKC)],
                dst_ref=rin_ref.at[pl.ds(peer * BLKC, BLKC)],
                send_sem=sy.at[peer],
                recv_sem=ry.at[peer],
                device_id=(peer,),
                device_id_type=pl.DeviceIdType.MESH,
            )
            recv.wait_recv()
            combine_block(peer)
        for rdma in y_rdmas:
            rdma.wait_send()

        @functools.partial(
            pl.run_scoped, exit_sem=pltpu.SemaphoreType.REGULAR
        )
        def _(exit_sem):
            for d in range(1, N_DEV):
                pl.semaphore_signal(
                    exit_sem, inc=1,
                    device_id=(lax.rem(my + d, N_DEV),),
                    device_id_type=pl.DeviceIdType.MESH,
                )
            pl.semaphore_wait(exit_sem, N_DEV - 1)

    return pl.pallas_call(
        body,
        out_shape=jax.ShapeDtypeStruct((n_tok, d_ff), jnp.float32),
        in_specs=[
            pl.BlockSpec(memory_space=pltpu.VMEM),
            pl.BlockSpec(memory_space=pltpu.VMEM),
            pl.BlockSpec(memory_space=pltpu.VMEM),
            pl.BlockSpec(memory_space=pltpu.VMEM),
            pl.BlockSpec(memory_space=pltpu.VMEM),
            pl.BlockSpec(memory_space=pltpu.VMEM),
            pl.BlockSpec(memory_space=pltpu.VMEM),
        ],
        out_specs=pl.BlockSpec(memory_space=pltpu.VMEM),
        scratch_shapes=[
            pltpu.VMEM((N_SLOTS, d_model), jnp.bfloat16),
            pltpu.VMEM((N_SLOTS, d_model), jnp.bfloat16),
            pltpu.VMEM((N_SLOTS, d_ff), jnp.bfloat16),
            pltpu.VMEM((N_SLOTS, d_ff), jnp.bfloat16),
            pltpu.VMEM((N_DEV * 8, 128), jnp.int32),
            pltpu.SemaphoreType.DMA((N_DEV,)),
            pltpu.SemaphoreType.DMA((N_DEV,)),
            pltpu.SemaphoreType.DMA((N_DEV,)),
            pltpu.SemaphoreType.DMA((N_DEV,)),
            pltpu.SemaphoreType.DMA((N_DEV,)),
            pltpu.SemaphoreType.DMA((N_DEV,)),
        ],
        compiler_params=pltpu.CompilerParams(
            collective_id=0,
            vmem_limit_bytes=100 * 1024 * 1024,
        ),
    )(x_bf, p_row, sid_row, sid_col, w_bf, sw_bf, cnt_in)
